# baseline (speedup 1.0000x reference)
import functools

import jax
import jax.numpy as jnp
from jax import lax
from jax.experimental import pallas as pl
from jax.experimental.pallas import tpu as pltpu
from jax.experimental.pallas import tpu_sc as plsc

N = 10000
D = 128
E = 320000
NPAD = 10240
EPAD = 327680
NC = 2
NS = 16
NW = NC * NS
CHUNK = 128
CPW = EPAD // NW // CHUNK
RPS = NPAD // NS
ZROWS = 160


def _mesh():
    return plsc.VectorSubcoreMesh(core_axis_name="c", subcore_axis_name="s")


QH = 40
NREF = CPW // QH


@functools.partial(
    pl.kernel,
    mesh=_mesh(),
    out_type=jax.ShapeDtypeStruct((NC * NPAD, D), jnp.float32),
    scratch_types=[
        pltpu.VMEM((QH, CHUNK), jnp.int32),
        pltpu.VMEM((QH, CHUNK), jnp.int32),
        pltpu.VMEM((2, CHUNK, D), jnp.float32),
        pltpu.VMEM_SHARED((NPAD, D), jnp.float32),
        pltpu.SemaphoreType.DMA,
        pltpu.SemaphoreType.DMA,
        pltpu.SemaphoreType.DMA,
        pltpu.SemaphoreType.DMA,
    ],
)
def _sc_agg(x_hbm, src_hbm, dst_hbm, out_hbm, src_v, dst_v, rbuf, acc,
            ga, gb, ta, tb):
    c = lax.axis_index("c")
    s = lax.axis_index("s")
    wid = s * NC + c
    gsem = (ga, gb)
    ssem = (ta, tb)
    zv = jnp.zeros((16,), jnp.float32)

    def _zrow(i, carry):
        for j in range(D // 16):
            rbuf[0, i, pl.ds(j * 16, 16)] = zv
        return carry

    lax.fori_loop(0, CHUNK, _zrow, 0)
    base = s * RPS
    for b in range(RPS // CHUNK):
        pltpu.sync_copy(rbuf.at[0], acc.at[pl.ds(base + b * CHUNK, CHUNK)])
    plsc.subcore_barrier()

    for q in range(NREF):
        hbase = wid * CPW + q * QH
        pltpu.sync_copy(src_hbm.at[pl.ds(hbase, QH)], src_v)
        pltpu.sync_copy(dst_hbm.at[pl.ds(hbase, QH)], dst_v)
        pltpu.async_copy(x_hbm.at[src_v.at[0]], rbuf.at[0], gsem[0])

        def _pair(t, carry):
            @pl.when(t > 0)
            def _():
                pltpu.make_async_copy(rbuf.at[1], acc.at[dst_v.at[2 * t - 1]],
                                      ssem[1]).wait()
            pltpu.async_copy(x_hbm.at[src_v.at[2 * t + 1]], rbuf.at[1],
                             gsem[1])
            pltpu.make_async_copy(x_hbm.at[src_v.at[2 * t]], rbuf.at[0],
                                  gsem[0]).wait()
            pltpu.async_copy(rbuf.at[0], acc.at[dst_v.at[2 * t]], ssem[0],
                             add=True)

            @pl.when(t + 1 < QH // 2)
            def _():
                pltpu.make_async_copy(rbuf.at[0], acc.at[dst_v.at[2 * t]],
                                      ssem[0]).wait()
                pltpu.async_copy(x_hbm.at[src_v.at[2 * t + 2]], rbuf.at[0],
                                 gsem[0])
            pltpu.make_async_copy(x_hbm.at[src_v.at[2 * t + 1]], rbuf.at[1],
                                  gsem[1]).wait()
            pltpu.async_copy(rbuf.at[1], acc.at[dst_v.at[2 * t + 1]], ssem[1],
                             add=True)
            return carry

        lax.fori_loop(0, QH // 2, _pair, 0)
        t = QH // 2 - 1
        pltpu.make_async_copy(rbuf.at[0], acc.at[dst_v.at[2 * t]],
                              ssem[0]).wait()
        pltpu.make_async_copy(rbuf.at[1], acc.at[dst_v.at[2 * t + 1]],
                              ssem[1]).wait()

    plsc.subcore_barrier()
    pltpu.sync_copy(acc.at[pl.ds(base, RPS)],
                    out_hbm.at[pl.ds(c * NPAD + base, RPS)])


@functools.partial(
    pl.kernel,
    mesh=_mesh(),
    out_type=jax.ShapeDtypeStruct((NC * NPAD, D), jnp.float32),
    scratch_types=[
        pltpu.VMEM((CPW, CHUNK), jnp.int32),
        pltpu.VMEM((CHUNK, D), jnp.float32),
        pltpu.VMEM_SHARED((NPAD, D), jnp.float32),
        pltpu.SemaphoreType.DMA,
        pltpu.SemaphoreType.DMA,
    ],
)
def _sc_deg(dst_hbm, out_hbm, dst_v, ones_v, acc, ta, tb):
    c = lax.axis_index("c")
    s = lax.axis_index("s")
    wid = s * NC + c
    ssem = (ta, tb)
    zv = jnp.zeros((16,), jnp.float32)
    ov = jnp.ones((16,), jnp.float32)

    def _fill(val):
        def _row(i, carry):
            for j in range(D // 16):
                ones_v[i, pl.ds(j * 16, 16)] = val
            return carry
        lax.fori_loop(0, CHUNK, _row, 0)

    _fill(zv)
    base = s * RPS
    for b in range(RPS // CHUNK):
        pltpu.sync_copy(ones_v, acc.at[pl.ds(base + b * CHUNK, CHUNK)])
    _fill(ov)
    pltpu.sync_copy(dst_hbm.at[pl.ds(wid * CPW, CPW)], dst_v)
    plsc.subcore_barrier()

    def _pair(t, carry):
        @pl.when(t > 0)
        def _():
            pltpu.make_async_copy(ones_v, acc.at[dst_v.at[2 * t - 2]],
                                  ssem[0]).wait()
            pltpu.make_async_copy(ones_v, acc.at[dst_v.at[2 * t - 1]],
                                  ssem[1]).wait()
        pltpu.async_copy(ones_v, acc.at[dst_v.at[2 * t]], ssem[0], add=True)
        pltpu.async_copy(ones_v, acc.at[dst_v.at[2 * t + 1]], ssem[1],
                         add=True)
        return carry

    lax.fori_loop(0, CPW // 2, _pair, 0)
    t = CPW // 2 - 1
    pltpu.make_async_copy(ones_v, acc.at[dst_v.at[2 * t]], ssem[0]).wait()
    pltpu.make_async_copy(ones_v, acc.at[dst_v.at[2 * t + 1]], ssem[1]).wait()
    plsc.subcore_barrier()
    pltpu.sync_copy(acc.at[pl.ds(base, RPS)],
                    out_hbm.at[pl.ds(c * NPAD + base, RPS)])


def _tc_layer_body(p_ref, d_ref, x_ref, wl_ref, bl_ref, wr_ref, o_ref):
    deg = jnp.maximum(d_ref[0, :, 0:1] + d_ref[1, :, 0:1], 1.0)
    mean = (p_ref[0] + p_ref[1]) / deg
    h = jnp.dot(mean, wl_ref[...], preferred_element_type=jnp.float32)
    h = h + bl_ref[...]
    h = h + jnp.dot(x_ref[...], wr_ref[...], preferred_element_type=jnp.float32)
    o_ref[...] = jnp.maximum(h, 0.0)


def _tc_final_body(p_ref, d_ref, x_ref, wl_ref, bl_ref, wr_ref,
                   wo_ref, bo_ref, o_ref):
    deg = jnp.maximum(d_ref[0, :, 0:1] + d_ref[1, :, 0:1], 1.0)
    mean = (p_ref[0] + p_ref[1]) / deg
    h = jnp.dot(mean, wl_ref[...], preferred_element_type=jnp.float32)
    h = h + bl_ref[...]
    h = h + jnp.dot(x_ref[...], wr_ref[...], preferred_element_type=jnp.float32)
    h = jnp.maximum(h, 0.0)
    o_ref[...] = jnp.dot(h, wo_ref[...], preferred_element_type=jnp.float32) + bo_ref[...]


_tc_layer = pl.pallas_call(
    _tc_layer_body,
    out_shape=jax.ShapeDtypeStruct((NPAD, D), jnp.float32),
)

_tc_final = pl.pallas_call(
    _tc_final_body,
    out_shape=jax.ShapeDtypeStruct((NPAD, D), jnp.float32),
)


def kernel(x, edge_index, Wl0, bl0, Wr0, Wl1, bl1, Wr1, Wl2, bl2, Wr2,
           Wl3, bl3, Wr3, Wl4, bl4, Wr4, Wl5, bl5, Wr5, Wout, bout):
    src = edge_index[0]
    dst = edge_index[1]
    pad = EPAD - E
    ar = jnp.arange(pad, dtype=jnp.int32)
    src_p = jnp.concatenate([src, ar % jnp.int32(N)])
    dst_p = jnp.concatenate([dst, jnp.int32(N) + ar % jnp.int32(NPAD - N)])
    src2d = src_p.reshape(EPAD // CHUNK, CHUNK)
    dst2d = dst_p.reshape(EPAD // CHUNK, CHUNK)
    xp = jnp.pad(x, ((0, NPAD - N), (0, 0)))

    dpart = _sc_deg(dst2d).reshape(NC, NPAD, D)[:, :, :16]

    layers = [(Wl0, bl0, Wr0), (Wl1, bl1, Wr1), (Wl2, bl2, Wr2),
              (Wl3, bl3, Wr3), (Wl4, bl4, Wr4)]
    h = xp
    for wl, bl, wr in layers:
        p = _sc_agg(h, src2d, dst2d).reshape(NC, NPAD, D)
        h = _tc_layer(p, dpart, h, wl, bl.reshape(1, D), wr)

    p = _sc_agg(h, src2d, dst2d).reshape(NC, NPAD, D)
    wo = jnp.pad(Wout, ((0, 0), (0, D - 1)))
    bo = jnp.pad(bout, (0, D - 1)).reshape(1, D)
    y = _tc_final(p, dpart, h, Wl5, bl5.reshape(1, D), Wr5, wo, bo)
    return y[:N, 0:1]


# speedup vs baseline: 1.0398x; 1.0398x over previous
import functools

import jax
import jax.numpy as jnp
from jax import lax
from jax.experimental import pallas as pl
from jax.experimental.pallas import tpu as pltpu
from jax.experimental.pallas import tpu_sc as plsc

N = 10000
D = 128
E = 320000
NPAD = 10240
EPAD = 327680
NC = 2
NS = 16
NW = NC * NS
CHUNK = 128
CPW = EPAD // NW // CHUNK
RPS = NPAD // NS
ZROWS = 160


def _mesh():
    return plsc.VectorSubcoreMesh(core_axis_name="c", subcore_axis_name="s")


CH2 = 64
NBUF = 4
CPW2 = EPAD // NW // CH2
QH = 40
NREF = CPW2 // QH


@functools.partial(
    pl.kernel,
    mesh=_mesh(),
    out_type=jax.ShapeDtypeStruct((NC * NPAD, D), jnp.float32),
    scratch_types=[
        pltpu.VMEM((QH, CH2), jnp.int32),
        pltpu.VMEM((QH, CH2), jnp.int32),
        pltpu.VMEM((NBUF, CH2, D), jnp.float32),
        pltpu.VMEM_SHARED((NPAD, D), jnp.float32),
        pltpu.SemaphoreType.DMA,
        pltpu.SemaphoreType.DMA,
        pltpu.SemaphoreType.DMA,
        pltpu.SemaphoreType.DMA,
    ],
)
def _sc_agg(x_hbm, src_hbm, dst_hbm, out_hbm, src_v, dst_v, rbuf, acc,
            s0, s1, s2, s3):
    c = lax.axis_index("c")
    s = lax.axis_index("s")
    wid = s * NC + c
    sems = (s0, s1, s2, s3)
    zv = jnp.zeros((16,), jnp.float32)

    def _zrow(i, carry):
        for j in range(D // 16):
            rbuf[0, i, pl.ds(j * 16, 16)] = zv
            rbuf[1, i, pl.ds(j * 16, 16)] = zv
        return carry

    lax.fori_loop(0, CH2, _zrow, 0)
    base = s * RPS
    for b in range(RPS // (2 * CH2)):
        pltpu.sync_copy(rbuf.at[0], acc.at[pl.ds(base + 2 * b * CH2, CH2)])
        pltpu.sync_copy(rbuf.at[1], acc.at[pl.ds(base + (2 * b + 1) * CH2, CH2)])
    plsc.subcore_barrier()

    for q in range(NREF):
        hbase = wid * CPW2 + q * QH
        pltpu.sync_copy(src_hbm.at[pl.ds(hbase, QH)], src_v)
        pltpu.sync_copy(dst_hbm.at[pl.ds(hbase, QH)], dst_v)
        for b in range(NBUF - 1):
            pltpu.async_copy(x_hbm.at[src_v.at[b]], rbuf.at[b], sems[b])

        def _group(t, carry):
            for k in range(NBUF):
                j = NBUF * t + k
                jn = j + NBUF - 1
                bn = (k + NBUF - 1) % NBUF
                pltpu.async_copy(x_hbm.at[src_v.at[jn]], rbuf.at[bn], sems[bn])
                pltpu.make_async_copy(x_hbm.at[src_v.at[j]], rbuf.at[k],
                                      sems[k]).wait()
                pltpu.sync_copy(rbuf.at[k], acc.at[dst_v.at[j]], add=True)
            return carry

        nfull = QH // NBUF - 1
        lax.fori_loop(0, nfull, _group, 0)
        for k in range(NBUF):
            j = NBUF * nfull + k
            jn = j + NBUF - 1
            bn = (k + NBUF - 1) % NBUF
            if jn < QH:
                pltpu.async_copy(x_hbm.at[src_v.at[jn]], rbuf.at[bn], sems[bn])
            pltpu.make_async_copy(x_hbm.at[src_v.at[j]], rbuf.at[k],
                                  sems[k]).wait()
            pltpu.sync_copy(rbuf.at[k], acc.at[dst_v.at[j]], add=True)

    plsc.subcore_barrier()
    pltpu.sync_copy(acc.at[pl.ds(base, RPS)],
                    out_hbm.at[pl.ds(c * NPAD + base, RPS)])


@functools.partial(
    pl.kernel,
    mesh=_mesh(),
    out_type=jax.ShapeDtypeStruct((NC * NPAD, D), jnp.float32),
    scratch_types=[
        pltpu.VMEM((CPW, CHUNK), jnp.int32),
        pltpu.VMEM((CHUNK, D), jnp.float32),
        pltpu.VMEM_SHARED((NPAD, D), jnp.float32),
    ],
)
def _sc_deg(dst_hbm, out_hbm, dst_v, ones_v, acc):
    c = lax.axis_index("c")
    s = lax.axis_index("s")
    wid = s * NC + c
    zv = jnp.zeros((16,), jnp.float32)
    ov = jnp.ones((16,), jnp.float32)

    def _fill(val):
        def _row(i, carry):
            for j in range(D // 16):
                ones_v[i, pl.ds(j * 16, 16)] = val
            return carry
        lax.fori_loop(0, CHUNK, _row, 0)

    _fill(zv)
    base = s * RPS
    for b in range(RPS // CHUNK):
        pltpu.sync_copy(ones_v, acc.at[pl.ds(base + b * CHUNK, CHUNK)])
    _fill(ov)
    pltpu.sync_copy(dst_hbm.at[pl.ds(wid * CPW, CPW)], dst_v)
    plsc.subcore_barrier()

    def _chunk(j, carry):
        pltpu.sync_copy(ones_v, acc.at[dst_v.at[j]], add=True)
        return carry

    lax.fori_loop(0, CPW, _chunk, 0)
    plsc.subcore_barrier()
    pltpu.sync_copy(acc.at[pl.ds(base, RPS)],
                    out_hbm.at[pl.ds(c * NPAD + base, RPS)])


def _tc_layer_body(p_ref, d_ref, x_ref, wl_ref, bl_ref, wr_ref, o_ref):
    deg = jnp.maximum(d_ref[0, :, 0:1] + d_ref[1, :, 0:1], 1.0)
    mean = (p_ref[0] + p_ref[1]) / deg
    h = jnp.dot(mean, wl_ref[...], preferred_element_type=jnp.float32)
    h = h + bl_ref[...]
    h = h + jnp.dot(x_ref[...], wr_ref[...], preferred_element_type=jnp.float32)
    o_ref[...] = jnp.maximum(h, 0.0)


def _tc_final_body(p_ref, d_ref, x_ref, wl_ref, bl_ref, wr_ref,
                   wo_ref, bo_ref, o_ref):
    deg = jnp.maximum(d_ref[0, :, 0:1] + d_ref[1, :, 0:1], 1.0)
    mean = (p_ref[0] + p_ref[1]) / deg
    h = jnp.dot(mean, wl_ref[...], preferred_element_type=jnp.float32)
    h = h + bl_ref[...]
    h = h + jnp.dot(x_ref[...], wr_ref[...], preferred_element_type=jnp.float32)
    h = jnp.maximum(h, 0.0)
    o_ref[...] = jnp.dot(h, wo_ref[...], preferred_element_type=jnp.float32) + bo_ref[...]


_tc_layer = pl.pallas_call(
    _tc_layer_body,
    out_shape=jax.ShapeDtypeStruct((NPAD, D), jnp.float32),
)

_tc_final = pl.pallas_call(
    _tc_final_body,
    out_shape=jax.ShapeDtypeStruct((NPAD, D), jnp.float32),
)


def kernel(x, edge_index, Wl0, bl0, Wr0, Wl1, bl1, Wr1, Wl2, bl2, Wr2,
           Wl3, bl3, Wr3, Wl4, bl4, Wr4, Wl5, bl5, Wr5, Wout, bout):
    src = edge_index[0]
    dst = edge_index[1]
    pad = EPAD - E
    ar = jnp.arange(pad, dtype=jnp.int32)
    src_p = jnp.concatenate([src, ar % jnp.int32(N)])
    dst_p = jnp.concatenate([dst, jnp.int32(N) + ar % jnp.int32(NPAD - N)])
    src2d = src_p.reshape(EPAD // CHUNK, CHUNK)
    dst2d = dst_p.reshape(EPAD // CHUNK, CHUNK)
    src64 = src_p.reshape(EPAD // CH2, CH2)
    dst64 = dst_p.reshape(EPAD // CH2, CH2)
    xp = jnp.pad(x, ((0, NPAD - N), (0, 0)))

    dpart = _sc_deg(dst2d).reshape(NC, NPAD, D)[:, :, :16]

    layers = [(Wl0, bl0, Wr0), (Wl1, bl1, Wr1), (Wl2, bl2, Wr2),
              (Wl3, bl3, Wr3), (Wl4, bl4, Wr4)]
    h = xp
    for wl, bl, wr in layers:
        p = _sc_agg(h, src64, dst64).reshape(NC, NPAD, D)
        h = _tc_layer(p, dpart, h, wl, bl.reshape(1, D), wr)

    p = _sc_agg(h, src64, dst64).reshape(NC, NPAD, D)
    wo = jnp.pad(Wout, ((0, 0), (0, D - 1)))
    bo = jnp.pad(bout, (0, D - 1)).reshape(1, D)
    y = _tc_final(p, dpart, h, Wl5, bl5.reshape(1, D), Wr5, wo, bo)
    return y[:N, 0:1]
